# Initial kernel scaffold; baseline (speedup 1.0000x reference)
import jax
import jax.numpy as jnp
from jax.experimental import pallas as pl


def kernel(x, adj, W1, b1, W2, b2):
    raise NotImplementedError("write your pallas kernel here")


# speedup vs baseline: 1.0566x; 1.0566x over previous
import functools

import jax
import jax.numpy as jnp
from jax.experimental import pallas as pl


def _layer_body(adj_ref, xin_ref, w_ref, b_ref, out_ref, *, activation):
    t = jnp.dot(adj_ref[...], xin_ref[...], preferred_element_type=jnp.float32)
    t = jnp.dot(t, w_ref[...], preferred_element_type=jnp.float32)
    t = t + b_ref[...]
    if activation == "relu":
        out_ref[...] = jnp.maximum(t, 0.0)
    else:
        m = jnp.max(t, axis=1, keepdims=True)
        s = t - m
        lse = jnp.log(jnp.sum(jnp.exp(s), axis=1, keepdims=True))
        out_ref[...] = s - lse


def _gcn_layer(adj, xin, w, b, activation, bm):
    n, _ = adj.shape
    d = w.shape[1]
    grid = (n // bm,)
    return pl.pallas_call(
        functools.partial(_layer_body, activation=activation),
        grid=grid,
        in_specs=[
            pl.BlockSpec((bm, n), lambda i: (i, 0)),
            pl.BlockSpec((n, xin.shape[1]), lambda i: (0, 0)),
            pl.BlockSpec(w.shape, lambda i: (0, 0)),
            pl.BlockSpec((1, d), lambda i: (0, 0)),
        ],
        out_specs=pl.BlockSpec((bm, d), lambda i: (i, 0)),
        out_shape=jax.ShapeDtypeStruct((n, d), jnp.float32),
    )(adj, xin, w, b)


def kernel(x, adj, W1, b1, W2, b2):
    n = adj.shape[0]
    bm = 200 if n % 200 == 0 else n
    b1r = b1.reshape(1, -1)
    b2r = b2.reshape(1, -1)
    h = _gcn_layer(adj, x, W1, b1r, "relu", bm)
    out = _gcn_layer(adj, h, W2, b2r, "log_softmax", bm)
    return out


# speedup vs baseline: 1.0942x; 1.0356x over previous
import functools

import jax
import jax.numpy as jnp
from jax.experimental import pallas as pl


def _layer_body(adj_ref, xin_ref, w_ref, b_ref, out_ref, *, activation):
    t = jnp.dot(adj_ref[...], xin_ref[...], preferred_element_type=jnp.float32)
    t = jnp.dot(t, w_ref[...], preferred_element_type=jnp.float32)
    t = t + b_ref[...]
    if activation == "relu":
        out_ref[...] = jnp.maximum(t, 0.0)
    else:
        m = jnp.max(t, axis=1, keepdims=True)
        s = t - m
        lse = jnp.log(jnp.sum(jnp.exp(s), axis=1, keepdims=True))
        out_ref[...] = s - lse


def _gcn_layer(adj, xin, w, b, activation, bm):
    n, _ = adj.shape
    d = w.shape[1]
    grid = (n // bm,)
    return pl.pallas_call(
        functools.partial(_layer_body, activation=activation),
        grid=grid,
        in_specs=[
            pl.BlockSpec((bm, n), lambda i: (i, 0)),
            pl.BlockSpec((n, xin.shape[1]), lambda i: (0, 0)),
            pl.BlockSpec(w.shape, lambda i: (0, 0)),
            pl.BlockSpec((1, d), lambda i: (0, 0)),
        ],
        out_specs=pl.BlockSpec((bm, d), lambda i: (i, 0)),
        out_shape=jax.ShapeDtypeStruct((n, d), jnp.float32),
    )(adj, xin, w, b)


def kernel(x, adj, W1, b1, W2, b2):
    n = adj.shape[0]
    bm = 400 if n % 400 == 0 else n
    b1r = b1.reshape(1, -1)
    b2r = b2.reshape(1, -1)
    h = _gcn_layer(adj, x, W1, b1r, "relu", bm)
    out = _gcn_layer(adj, h, W2, b2r, "log_softmax", bm)
    return out


# speedup vs baseline: 1.0948x; 1.0006x over previous
import functools

import jax
import jax.numpy as jnp
from jax.experimental import pallas as pl
from jax.experimental.pallas import tpu as pltpu


def _layer_body(adj_ref, xin_ref, w_ref, b_ref, out_ref, *, activation):
    t = jnp.dot(adj_ref[...].astype(jnp.bfloat16),
                xin_ref[...].astype(jnp.bfloat16),
                preferred_element_type=jnp.float32)
    t = jnp.dot(t, w_ref[...], preferred_element_type=jnp.float32)
    t = t + b_ref[...]
    if activation == "relu":
        out_ref[...] = jnp.maximum(t, 0.0)
    else:
        m = jnp.max(t, axis=1, keepdims=True)
        s = t - m
        lse = jnp.log(jnp.sum(jnp.exp(s), axis=1, keepdims=True))
        out_ref[...] = s - lse


def _gcn_layer(adj, xin, w, b, activation, bm):
    n, _ = adj.shape
    d = w.shape[1]
    grid = (n // bm,)
    return pl.pallas_call(
        functools.partial(_layer_body, activation=activation),
        grid=grid,
        in_specs=[
            pl.BlockSpec((bm, n), lambda i: (i, 0)),
            pl.BlockSpec((n, xin.shape[1]), lambda i: (0, 0)),
            pl.BlockSpec(w.shape, lambda i: (0, 0)),
            pl.BlockSpec((1, d), lambda i: (0, 0)),
        ],
        out_specs=pl.BlockSpec((bm, d), lambda i: (i, 0)),
        out_shape=jax.ShapeDtypeStruct((n, d), jnp.float32),
        compiler_params=pltpu.CompilerParams(vmem_limit_bytes=115 * 1024 * 1024),
    )(adj, xin, w, b)


def kernel(x, adj, W1, b1, W2, b2):
    n = adj.shape[0]
    bm = 400 if n % 400 == 0 else n
    b1r = b1.reshape(1, -1)
    b2r = b2.reshape(1, -1)
    h = _gcn_layer(adj, x, W1, b1r, "relu", bm)
    out = _gcn_layer(adj, h, W2, b2r, "log_softmax", bm)
    return out
